# baseline (speedup 1.0000x reference)
import jax
import jax.numpy as jnp
from jax import lax
from jax.experimental import pallas as pl
from jax.experimental.pallas import tpu as pltpu
from jax.experimental.pallas import tpu_sc as plsc

N_NODES = 10000
D = 128
N_EDGES = 320000

N_TILES = 32
CHUNK = 80
BLK = 21
SC0_BLOCKS = 4
SC1_BLOCKS = 8
NBLK_MAX = max(SC0_BLOCKS, SC1_BLOCKS)
EDGES_PER_BLOCK = BLK * CHUNK
ROWS_PER_TILE = N_NODES // 16


def _sc_agg_body(x_hbm, src_hbm, dst_hbm, w_hbm, out_hbm,
                 idx_a, idx_b, dst_a, dst_b, w_a, w_b,
                 r0, r1, r2, accum_sh, sem0, sem1, sem2):
    c = lax.axis_index("c")
    s = lax.axis_index("s")
    wid = c * 16 + s
    base_row = s * ROWS_PER_TILE
    rows = (r0, r1, r2)
    sems = (sem0, sem1, sem2)
    idx_st = (idx_a, idx_b)
    dst_st = (dst_a, dst_b)
    w_st = (w_a, w_b)
    nblk_self = jnp.where(c == 0, SC0_BLOCKS, SC1_BLOCKS)
    nchunk_self = nblk_self * BLK

    def zero_row(i, carry):
        for j in range(8):
            r0[i, pl.ds(j * 16, 16)] = jnp.zeros((16,), jnp.float32)
        return carry
    lax.fori_loop(0, CHUNK, zero_row, 0)
    for t in range(ROWS_PER_TILE // CHUNK):
        pltpu.sync_copy(r0, accum_sh.at[pl.ds(base_row + t * CHUNK, CHUNK)])
    rem = ROWS_PER_TILE % CHUNK
    if rem:
        pltpu.sync_copy(
            r0.at[pl.ds(0, rem)],
            accum_sh.at[pl.ds(base_row + (ROWS_PER_TILE // CHUNK) * CHUNK, rem)])
    plsc.subcore_barrier()

    def refill(blk_static):
        sb = blk_static % 2
        pltpu.sync_copy(src_hbm.at[wid, blk_static], idx_st[sb])
        pltpu.sync_copy(dst_hbm.at[wid, blk_static], dst_st[sb])
        pltpu.sync_copy(w_hbm.at[wid, blk_static], w_st[sb])

    def scale_dyn(pos, par, m):
        buf = rows[m]

        def grp(g, carry):
            base = g * 16
            wa16 = w_a[pos, pl.ds(base, 16)]
            wb16 = w_b[pos, pl.ds(base, 16)]
            w16 = jnp.where(par == 0, wa16, wb16)
            for r in range(16):
                wv = jnp.full((16,), w16[r], jnp.float32)
                for j in range(8):
                    sl = pl.ds(j * 16, 16)
                    buf[base + r, sl] = buf[base + r, sl] * wv
            return carry
        lax.fori_loop(0, CHUNK // 16, grp, 0)

    refill(0)
    pltpu.async_copy(x_hbm.at[idx_a.at[0]], rows[0], sems[0])
    pltpu.async_copy(x_hbm.at[idx_a.at[1]], rows[1], sems[1])

    def round_body(q, carry):
        k0 = 3 * q
        for b in range(3):
            k = k0 + b
            m = b
            blk = k // BLK
            par = lax.rem(blk, 2)
            pos = lax.rem(k, BLK)

            pltpu.make_async_copy(
                x_hbm.at[idx_a.at[0]], rows[m], sems[m]).wait()
            scale_dyn(pos, par, m)

            m_prev = (b + 2) % 3

            @pl.when(k > 0)
            def _():
                pltpu.make_async_copy(
                    rows[m_prev], accum_sh.at[dst_a.at[0]],
                    sems[m_prev]).wait()

            @pl.when(par == 0)
            def _():
                pltpu.async_copy(rows[m], accum_sh.at[dst_a.at[pos]],
                                 sems[m], add=True)

            @pl.when(par == 1)
            def _():
                pltpu.async_copy(rows[m], accum_sh.at[dst_b.at[pos]],
                                 sems[m], add=True)

            k2 = k + 2
            pos2 = lax.rem(k2, BLK)
            par2 = lax.rem(k2 // BLK, 2)

            @pl.when((k2 < nchunk_self) & (par2 == 0))
            def _():
                pltpu.async_copy(x_hbm.at[idx_a.at[pos2]], rows[m_prev],
                                 sems[m_prev])

            @pl.when((k2 < nchunk_self) & (par2 == 1))
            def _():
                pltpu.async_copy(x_hbm.at[idx_b.at[pos2]], rows[m_prev],
                                 sems[m_prev])

            if b == 0:
                nblk = blk + 1
                do = (pos == 0) & (nblk < nblk_self)
                rp = lax.rem(nblk, 2)

                @pl.when(do & (rp == 0))
                def _():
                    pltpu.sync_copy(src_hbm.at[wid, nblk], idx_a)
                    pltpu.sync_copy(dst_hbm.at[wid, nblk], dst_a)
                    pltpu.sync_copy(w_hbm.at[wid, nblk], w_a)

                @pl.when(do & (rp == 1))
                def _():
                    pltpu.sync_copy(src_hbm.at[wid, nblk], idx_b)
                    pltpu.sync_copy(dst_hbm.at[wid, nblk], dst_b)
                    pltpu.sync_copy(w_hbm.at[wid, nblk], w_b)
        return carry
    lax.fori_loop(0, nblk_self * (BLK // 3), round_body, 0)

    pltpu.make_async_copy(
        rows[2], accum_sh.at[dst_a.at[0]], sems[2]).wait()
    plsc.subcore_barrier()
    pltpu.sync_copy(accum_sh.at[pl.ds(base_row, ROWS_PER_TILE)],
                    out_hbm.at[c, s])


_sc_aggregate = pl.kernel(
    _sc_agg_body,
    out_type=jax.ShapeDtypeStruct((2, 16, ROWS_PER_TILE, D), jnp.float32),
    mesh=plsc.VectorSubcoreMesh(core_axis_name="c", subcore_axis_name="s"),
    scratch_types=[
        pltpu.VMEM((BLK, CHUNK), jnp.int32),
        pltpu.VMEM((BLK, CHUNK), jnp.int32),
        pltpu.VMEM((BLK, CHUNK), jnp.int32),
        pltpu.VMEM((BLK, CHUNK), jnp.int32),
        pltpu.VMEM((BLK, CHUNK), jnp.float32),
        pltpu.VMEM((BLK, CHUNK), jnp.float32),
        pltpu.VMEM((CHUNK, D), jnp.float32),
        pltpu.VMEM((CHUNK, D), jnp.float32),
        pltpu.VMEM((CHUNK, D), jnp.float32),
        pltpu.VMEM_SHARED((N_NODES, D), jnp.float32),
        pltpu.SemaphoreType.DMA,
        pltpu.SemaphoreType.DMA,
        pltpu.SemaphoreType.DMA,
    ],
)


def _relu_combine_body(p_ref, b1_ref, o_ref):
    o_ref[...] = jnp.maximum(p_ref[0] + p_ref[1] + b1_ref[...], 0.0)


def _matmul_combine_body(q_ref, w2_ref, b2_ref, o_ref):
    agg = q_ref[0] + q_ref[1]
    o_ref[...] = (
        jnp.dot(agg, w2_ref[...], preferred_element_type=jnp.float32)
        + b2_ref[...])


_TC_BLOCK = 1000


def _relu_combine(p, b1):
    return pl.pallas_call(
        _relu_combine_body,
        grid=(N_NODES // _TC_BLOCK,),
        in_specs=[
            pl.BlockSpec((2, _TC_BLOCK, D), lambda i: (0, i, 0)),
            pl.BlockSpec((1, D), lambda i: (0, 0)),
        ],
        out_specs=pl.BlockSpec((_TC_BLOCK, D), lambda i: (i, 0)),
        out_shape=jax.ShapeDtypeStruct((N_NODES, D), jnp.float32),
    )(p, b1.reshape(1, D))


def _matmul_combine(q, W2, b2):
    return pl.pallas_call(
        _matmul_combine_body,
        grid=(N_NODES // _TC_BLOCK,),
        in_specs=[
            pl.BlockSpec((2, _TC_BLOCK, D), lambda i: (0, i, 0)),
            pl.BlockSpec((D, D), lambda i: (0, 0)),
            pl.BlockSpec((1, D), lambda i: (0, 0)),
        ],
        out_specs=pl.BlockSpec((_TC_BLOCK, D), lambda i: (i, 0)),
        out_shape=jax.ShapeDtypeStruct((N_NODES, D), jnp.float32),
    )(q, W2, b2.reshape(1, D))


def _prep_edges(edge_index, edge_weight):
    e0 = 16 * SC0_BLOCKS * EDGES_PER_BLOCK
    e1cap = 16 * SC1_BLOCKS * EDGES_PER_BLOCK

    def split(arr):
        a0 = arr[:e0].reshape(16, SC0_BLOCKS, BLK, CHUNK)
        a0 = jnp.pad(a0, ((0, 0), (0, NBLK_MAX - SC0_BLOCKS), (0, 0), (0, 0)))
        a1 = jnp.pad(arr[e0:], (0, e1cap - (N_EDGES - e0)))
        a1 = a1.reshape(16, SC1_BLOCKS, BLK, CHUNK)
        a1 = jnp.pad(a1, ((0, 0), (0, NBLK_MAX - SC1_BLOCKS), (0, 0), (0, 0)))
        return jnp.concatenate([a0, a1], axis=0)

    return (split(edge_index[0].astype(jnp.int32)),
            split(edge_index[1].astype(jnp.int32)),
            split(edge_weight))


@jax.jit
def kernel(x, edge_index1, edge_weight1, edge_index2, edge_weight2, W2, b1, b2):
    src1, dst1, w1 = _prep_edges(edge_index1, edge_weight1)
    src2, dst2, w2 = _prep_edges(edge_index2, edge_weight2)

    p1 = _sc_aggregate(x, src1, dst1, w1).reshape(2, N_NODES, D)
    h = _relu_combine(p1, b1)
    p2 = _sc_aggregate(h, src2, dst2, w2).reshape(2, N_NODES, D)
    return _matmul_combine(p2, W2, b2)


# speedup vs baseline: 1.2116x; 1.2116x over previous
import jax
import jax.numpy as jnp
from jax import lax
from jax.experimental import pallas as pl
from jax.experimental.pallas import tpu as pltpu
from jax.experimental.pallas import tpu_sc as plsc

N_NODES = 10000
D = 128
N_EDGES = 320000

N_TILES = 32
CHUNK = 80
BLK = 21
SC0_BLOCKS = 8
SC1_BLOCKS = 4
NBLK_MAX = max(SC0_BLOCKS, SC1_BLOCKS)
EDGES_PER_BLOCK = BLK * CHUNK
ROWS_PER_TILE = N_NODES // 16


def _sc_agg_body(x_hbm, src_hbm, dst_hbm, w_hbm, out_hbm,
                 idx_a, idx_b, dst_a, dst_b, w_a, w_b,
                 r0, r1, r2, accum_sh, sem0, sem1, sem2):
    c = lax.axis_index("c")
    s = lax.axis_index("s")
    wid = c * 16 + s
    base_row = s * ROWS_PER_TILE
    rows = (r0, r1, r2)
    sems = (sem0, sem1, sem2)
    idx_st = (idx_a, idx_b)
    dst_st = (dst_a, dst_b)
    w_st = (w_a, w_b)
    nblk_self = jnp.where(c == 0, SC0_BLOCKS, SC1_BLOCKS)
    nchunk_self = nblk_self * BLK

    def zero_row(i, carry):
        for j in range(8):
            r0[i, pl.ds(j * 16, 16)] = jnp.zeros((16,), jnp.float32)
        return carry
    lax.fori_loop(0, CHUNK, zero_row, 0)
    for t in range(ROWS_PER_TILE // CHUNK):
        pltpu.sync_copy(r0, accum_sh.at[pl.ds(base_row + t * CHUNK, CHUNK)])
    rem = ROWS_PER_TILE % CHUNK
    if rem:
        pltpu.sync_copy(
            r0.at[pl.ds(0, rem)],
            accum_sh.at[pl.ds(base_row + (ROWS_PER_TILE // CHUNK) * CHUNK, rem)])
    plsc.subcore_barrier()

    def refill(blk_static):
        sb = blk_static % 2
        pltpu.sync_copy(src_hbm.at[wid, blk_static], idx_st[sb])
        pltpu.sync_copy(dst_hbm.at[wid, blk_static], dst_st[sb])
        pltpu.sync_copy(w_hbm.at[wid, blk_static], w_st[sb])

    def scale_dyn(pos, par, m):
        buf = rows[m]

        def grp(g, carry):
            base = g * 16
            wa16 = w_a[pos, pl.ds(base, 16)]
            wb16 = w_b[pos, pl.ds(base, 16)]
            w16 = jnp.where(par == 0, wa16, wb16)
            for r in range(16):
                wv = jnp.full((16,), w16[r], jnp.float32)
                for j in range(8):
                    sl = pl.ds(j * 16, 16)
                    buf[base + r, sl] = buf[base + r, sl] * wv
            return carry
        lax.fori_loop(0, CHUNK // 16, grp, 0)

    refill(0)
    pltpu.async_copy(x_hbm.at[idx_a.at[0]], rows[0], sems[0])
    pltpu.async_copy(x_hbm.at[idx_a.at[1]], rows[1], sems[1])

    def round_body(q, carry):
        k0 = 3 * q
        for b in range(3):
            k = k0 + b
            m = b
            blk = k // BLK
            par = lax.rem(blk, 2)
            pos = lax.rem(k, BLK)

            pltpu.make_async_copy(
                x_hbm.at[idx_a.at[0]], rows[m], sems[m]).wait()
            scale_dyn(pos, par, m)

            m_prev = (b + 2) % 3

            @pl.when(k > 0)
            def _():
                pltpu.make_async_copy(
                    rows[m_prev], accum_sh.at[dst_a.at[0]],
                    sems[m_prev]).wait()

            @pl.when(par == 0)
            def _():
                pltpu.async_copy(rows[m], accum_sh.at[dst_a.at[pos]],
                                 sems[m], add=True)

            @pl.when(par == 1)
            def _():
                pltpu.async_copy(rows[m], accum_sh.at[dst_b.at[pos]],
                                 sems[m], add=True)

            k2 = k + 2
            pos2 = lax.rem(k2, BLK)
            par2 = lax.rem(k2 // BLK, 2)

            @pl.when((k2 < nchunk_self) & (par2 == 0))
            def _():
                pltpu.async_copy(x_hbm.at[idx_a.at[pos2]], rows[m_prev],
                                 sems[m_prev])

            @pl.when((k2 < nchunk_self) & (par2 == 1))
            def _():
                pltpu.async_copy(x_hbm.at[idx_b.at[pos2]], rows[m_prev],
                                 sems[m_prev])

            if b == 0:
                nblk = blk + 1
                do = (pos == 0) & (nblk < nblk_self)
                rp = lax.rem(nblk, 2)

                @pl.when(do & (rp == 0))
                def _():
                    pltpu.sync_copy(src_hbm.at[wid, nblk], idx_a)
                    pltpu.sync_copy(dst_hbm.at[wid, nblk], dst_a)
                    pltpu.sync_copy(w_hbm.at[wid, nblk], w_a)

                @pl.when(do & (rp == 1))
                def _():
                    pltpu.sync_copy(src_hbm.at[wid, nblk], idx_b)
                    pltpu.sync_copy(dst_hbm.at[wid, nblk], dst_b)
                    pltpu.sync_copy(w_hbm.at[wid, nblk], w_b)
        return carry
    lax.fori_loop(0, nblk_self * (BLK // 3), round_body, 0)

    pltpu.make_async_copy(
        rows[2], accum_sh.at[dst_a.at[0]], sems[2]).wait()
    plsc.subcore_barrier()
    pltpu.sync_copy(accum_sh.at[pl.ds(base_row, ROWS_PER_TILE)],
                    out_hbm.at[c, s])


_sc_aggregate = pl.kernel(
    _sc_agg_body,
    out_type=jax.ShapeDtypeStruct((2, 16, ROWS_PER_TILE, D), jnp.float32),
    mesh=plsc.VectorSubcoreMesh(core_axis_name="c", subcore_axis_name="s"),
    scratch_types=[
        pltpu.VMEM((BLK, CHUNK), jnp.int32),
        pltpu.VMEM((BLK, CHUNK), jnp.int32),
        pltpu.VMEM((BLK, CHUNK), jnp.int32),
        pltpu.VMEM((BLK, CHUNK), jnp.int32),
        pltpu.VMEM((BLK, CHUNK), jnp.float32),
        pltpu.VMEM((BLK, CHUNK), jnp.float32),
        pltpu.VMEM((CHUNK, D), jnp.float32),
        pltpu.VMEM((CHUNK, D), jnp.float32),
        pltpu.VMEM((CHUNK, D), jnp.float32),
        pltpu.VMEM_SHARED((N_NODES, D), jnp.float32),
        pltpu.SemaphoreType.DMA,
        pltpu.SemaphoreType.DMA,
        pltpu.SemaphoreType.DMA,
    ],
)


def _relu_combine_body(p_ref, b1_ref, o_ref):
    o_ref[...] = jnp.maximum(p_ref[0] + p_ref[1] + b1_ref[...], 0.0)


def _matmul_combine_body(q_ref, w2_ref, b2_ref, o_ref):
    agg = q_ref[0] + q_ref[1]
    o_ref[...] = (
        jnp.dot(agg, w2_ref[...], preferred_element_type=jnp.float32)
        + b2_ref[...])


_TC_BLOCK = 1000


def _relu_combine(p, b1):
    return pl.pallas_call(
        _relu_combine_body,
        grid=(N_NODES // _TC_BLOCK,),
        in_specs=[
            pl.BlockSpec((2, _TC_BLOCK, D), lambda i: (0, i, 0)),
            pl.BlockSpec((1, D), lambda i: (0, 0)),
        ],
        out_specs=pl.BlockSpec((_TC_BLOCK, D), lambda i: (i, 0)),
        out_shape=jax.ShapeDtypeStruct((N_NODES, D), jnp.float32),
    )(p, b1.reshape(1, D))


def _matmul_combine(q, W2, b2):
    return pl.pallas_call(
        _matmul_combine_body,
        grid=(N_NODES // _TC_BLOCK,),
        in_specs=[
            pl.BlockSpec((2, _TC_BLOCK, D), lambda i: (0, i, 0)),
            pl.BlockSpec((D, D), lambda i: (0, 0)),
            pl.BlockSpec((1, D), lambda i: (0, 0)),
        ],
        out_specs=pl.BlockSpec((_TC_BLOCK, D), lambda i: (i, 0)),
        out_shape=jax.ShapeDtypeStruct((N_NODES, D), jnp.float32),
    )(q, W2, b2.reshape(1, D))


def _prep_edges(edge_index, edge_weight):
    e0 = 16 * SC0_BLOCKS * EDGES_PER_BLOCK
    e1cap = 16 * SC1_BLOCKS * EDGES_PER_BLOCK

    def split(arr):
        a0 = arr[:e0].reshape(16, SC0_BLOCKS, BLK, CHUNK)
        a0 = jnp.pad(a0, ((0, 0), (0, NBLK_MAX - SC0_BLOCKS), (0, 0), (0, 0)))
        a1 = jnp.pad(arr[e0:], (0, e1cap - (N_EDGES - e0)))
        a1 = a1.reshape(16, SC1_BLOCKS, BLK, CHUNK)
        a1 = jnp.pad(a1, ((0, 0), (0, NBLK_MAX - SC1_BLOCKS), (0, 0), (0, 0)))
        return jnp.concatenate([a0, a1], axis=0)

    return (split(edge_index[0].astype(jnp.int32)),
            split(edge_index[1].astype(jnp.int32)),
            split(edge_weight))


@jax.jit
def kernel(x, edge_index1, edge_weight1, edge_index2, edge_weight2, W2, b1, b2):
    src1, dst1, w1 = _prep_edges(edge_index1, edge_weight1)
    src2, dst2, w2 = _prep_edges(edge_index2, edge_weight2)

    p1 = _sc_aggregate(x, src1, dst1, w1).reshape(2, N_NODES, D)
    h = _relu_combine(p1, b1)
    p2 = _sc_aggregate(h, src2, dst2, w2).reshape(2, N_NODES, D)
    return _matmul_combine(p2, W2, b2)
